# Initial kernel scaffold; baseline (speedup 1.0000x reference)
import jax
import jax.numpy as jnp
from jax.experimental import pallas as pl


def kernel(x, edge_index, edge_type, c0_lin_w, c0_film_w, c0_film_b, c0_skip_w, c0_sfilm_w, c0_sfilm_b, bn_g, bn_b, c1_lin_w, c1_film_w, c1_film_b, c1_skip_w, c1_sfilm_w, c1_sfilm_b):
    raise NotImplementedError("write your pallas kernel here")


# speedup vs baseline: 16.3133x; 16.3133x over previous
import functools

import jax
import jax.numpy as jnp
from jax import lax
from jax.experimental import pallas as pl
from jax.experimental.pallas import tpu as pltpu
from jax.experimental.pallas import tpu_sc as plsc

N = 10000
E = 320000
IN_C = 128
HID = 128
OUT_C = 64
R = 4
RN = R * N

NC = 2
NS = 16
NW = NC * NS
EPW = E // NW
C = 80
SI = 2000
NSI = EPW // SI
NCI = SI // C

RNP = 40960
CNT_SLICE = RNP // NS

NROW = 10240
RPS = NROW // NS

_mesh = plsc.VectorSubcoreMesh(core_axis_name="c", subcore_axis_name="s")

_f32 = jnp.float32
_bf16 = jnp.bfloat16


def _zero_vec_loop(ref, rows, width):
    def body(i, _):
        for u in range(width // 16):
            ref[i, pl.ds(u * 16, 16)] = jnp.zeros((16,), _f32)
        return 0
    lax.fori_loop(0, rows, body, 0, unroll=False)


@functools.partial(
    pl.kernel,
    out_type=jax.ShapeDtypeStruct((NC, RNP), _f32),
    mesh=_mesh,
    scratch_types=[
        pltpu.VMEM_SHARED((RNP,), _f32),
        pltpu.VMEM((SI,), jnp.int32),
        pltpu.VMEM((SI,), jnp.int32),
        pltpu.VMEM((C,), jnp.int32),
        pltpu.VMEM((C,), _f32),
        pltpu.VMEM((CNT_SLICE,), _f32),
    ],
)
def _sc_counts(dst_hbm, typ_hbm, out_hbm, cnt_sh, dstv, typv, idxv, onesv, zb):
    c = lax.axis_index("c")
    s = lax.axis_index("s")
    wid = c * NS + s

    def zb_body(i, _):
        zb[pl.ds(i * 16, 16)] = jnp.zeros((16,), _f32)
        return 0
    lax.fori_loop(0, CNT_SLICE // 16, zb_body, 0, unroll=False)
    for j in range(C // 16):
        onesv[pl.ds(j * 16, 16)] = jnp.ones((16,), _f32)

    pltpu.sync_copy(zb, cnt_sh.at[pl.ds(s * CNT_SLICE, CNT_SLICE)])
    plsc.subcore_barrier()

    base = wid * EPW

    def superchunk(m, _):
        off = base + m * SI
        pltpu.sync_copy(dst_hbm.at[pl.ds(off, SI)], dstv)
        pltpu.sync_copy(typ_hbm.at[pl.ds(off, SI)], typv)

        def chunk(k, _):
            for j in range(C // 16):
                d16 = dstv[pl.ds(k * C + j * 16, 16)]
                r16 = typv[pl.ds(k * C + j * 16, 16)]
                idxv[pl.ds(j * 16, 16)] = r16 * N + d16
            pltpu.sync_copy(onesv, cnt_sh.at[idxv], add=True)
            return 0
        lax.fori_loop(0, NCI, chunk, 0, unroll=False)
        return 0
    lax.fori_loop(0, NSI, superchunk, 0, unroll=False)

    plsc.subcore_barrier()
    pltpu.sync_copy(cnt_sh.at[pl.ds(s * CNT_SLICE, CNT_SLICE)],
                    out_hbm.at[c, pl.ds(s * CNT_SLICE, CNT_SLICE)])


@functools.partial(
    pl.kernel,
    out_type=jax.ShapeDtypeStruct((NC, NROW, HID), _f32),
    mesh=_mesh,
    scratch_types=[
        pltpu.VMEM_SHARED((NROW, HID), _f32),
        pltpu.VMEM((SI,), jnp.int32),
        pltpu.VMEM((SI,), jnp.int32),
        pltpu.VMEM((SI,), jnp.int32),
        pltpu.VMEM((C,), jnp.int32),
        pltpu.VMEM((C,), jnp.int32),
        pltpu.VMEM((C,), jnp.int32),
        pltpu.VMEM((C, HID), _f32),
        pltpu.VMEM((C, 2 * HID), _f32),
        pltpu.SemaphoreType.DMA,
        pltpu.SemaphoreType.DMA,
    ],
)
def _sc_edge0(xl_hbm, film_hbm, src_hbm, dst_hbm, typ_hbm, out_hbm,
              acc_sh, srcv, dstv, typv, isv, idv, sctv, xlb, fb,
              sem1, sem2):
    c = lax.axis_index("c")
    s = lax.axis_index("s")
    wid = c * NS + s
    row0 = s * RPS

    _zero_vec_loop(xlb, C, HID)
    for j in range(RPS // C):
        pltpu.sync_copy(xlb, acc_sh.at[pl.ds(row0 + j * C, C), :])
    plsc.subcore_barrier()

    base = wid * EPW

    def superchunk(m, _):
        off = base + m * SI
        pltpu.sync_copy(src_hbm.at[pl.ds(off, SI)], srcv)
        pltpu.sync_copy(dst_hbm.at[pl.ds(off, SI)], dstv)
        pltpu.sync_copy(typ_hbm.at[pl.ds(off, SI)], typv)

        def chunk(k, _):
            for j in range(C // 16):
                s16 = srcv[pl.ds(k * C + j * 16, 16)]
                d16 = dstv[pl.ds(k * C + j * 16, 16)]
                r16 = typv[pl.ds(k * C + j * 16, 16)]
                isv[pl.ds(j * 16, 16)] = r16 * N + s16
                idv[pl.ds(j * 16, 16)] = r16 * N + d16
                sctv[pl.ds(j * 16, 16)] = d16
            cp1 = pltpu.async_copy(xl_hbm.at[isv], xlb, sem1)
            cp2 = pltpu.async_copy(film_hbm.at[idv], fb, sem2)
            cp1.wait()
            cp2.wait()

            def row(i, _):
                for u in range(HID // 16):
                    x16 = xlb[i, pl.ds(u * 16, 16)]
                    b16 = fb[i, pl.ds(u * 16, 16)]
                    g16 = fb[i, pl.ds(HID + u * 16, 16)]
                    xlb[i, pl.ds(u * 16, 16)] = jnp.maximum(
                        g16 * x16 + b16, 0.0)
                return 0
            lax.fori_loop(0, C, row, 0, unroll=False)

            pltpu.sync_copy(xlb, acc_sh.at[sctv], add=True)
            return 0
        lax.fori_loop(0, NCI, chunk, 0, unroll=False)
        return 0
    lax.fori_loop(0, NSI, superchunk, 0, unroll=False)

    plsc.subcore_barrier()
    for j in range(RPS // 128):
        pltpu.sync_copy(acc_sh.at[pl.ds(row0 + j * 128, 128), :],
                        out_hbm.at[c, pl.ds(row0 + j * 128, 128), :])


@functools.partial(
    pl.kernel,
    out_type=jax.ShapeDtypeStruct((NC, NROW, 2 * OUT_C), _f32),
    mesh=_mesh,
    scratch_types=[
        pltpu.VMEM_SHARED((NROW, 2 * OUT_C), _f32),
        pltpu.VMEM((SI,), jnp.int32),
        pltpu.VMEM((SI,), jnp.int32),
        pltpu.VMEM((SI,), jnp.int32),
        pltpu.VMEM((C,), jnp.int32),
        pltpu.VMEM((C,), jnp.int32),
        pltpu.VMEM((C,), jnp.int32),
        pltpu.VMEM((C, 2 * OUT_C), _f32),
        pltpu.VMEM((C, 2 * OUT_C), _f32),
        pltpu.VMEM((C, 2 * OUT_C), _f32),
        pltpu.SemaphoreType.DMA,
        pltpu.SemaphoreType.DMA,
    ],
)
def _sc_edge1(t_hbm, src_hbm, dst_hbm, typ_hbm, out_hbm,
              acc_sh, srcv, dstv, typv, isv, idv, sctv, xlb, gb, mb,
              sem1, sem2):
    c = lax.axis_index("c")
    s = lax.axis_index("s")
    wid = c * NS + s
    row0 = s * RPS

    _zero_vec_loop(mb, C, 2 * OUT_C)
    for j in range(RPS // C):
        pltpu.sync_copy(mb, acc_sh.at[pl.ds(row0 + j * C, C), :])
    plsc.subcore_barrier()

    base = wid * EPW

    def superchunk(m, _):
        off = base + m * SI
        pltpu.sync_copy(src_hbm.at[pl.ds(off, SI)], srcv)
        pltpu.sync_copy(dst_hbm.at[pl.ds(off, SI)], dstv)
        pltpu.sync_copy(typ_hbm.at[pl.ds(off, SI)], typv)

        def chunk(k, _):
            for j in range(C // 16):
                s16 = srcv[pl.ds(k * C + j * 16, 16)]
                d16 = dstv[pl.ds(k * C + j * 16, 16)]
                r16 = typv[pl.ds(k * C + j * 16, 16)]
                isv[pl.ds(j * 16, 16)] = r16 * N + s16
                idv[pl.ds(j * 16, 16)] = r16 * N + d16
                sctv[pl.ds(j * 16, 16)] = d16
            cp1 = pltpu.async_copy(t_hbm.at[isv], xlb, sem1)
            cp2 = pltpu.async_copy(t_hbm.at[idv], gb, sem2)
            cp1.wait()
            cp2.wait()

            def row(i, _):
                for u in range(OUT_C // 16):
                    x16 = xlb[i, pl.ds(u * 16, 16)]
                    g16 = gb[i, pl.ds(OUT_C + u * 16, 16)]
                    mb[i, pl.ds(u * 16, 16)] = g16 * x16
                return 0
            lax.fori_loop(0, C, row, 0, unroll=False)

            pltpu.sync_copy(mb, acc_sh.at[sctv], add=True)
            return 0
        lax.fori_loop(0, NCI, chunk, 0, unroll=False)
        return 0
    lax.fori_loop(0, NSI, superchunk, 0, unroll=False)

    plsc.subcore_barrier()
    for j in range(RPS // 128):
        pltpu.sync_copy(acc_sh.at[pl.ds(row0 + j * 128, 128), :],
                        out_hbm.at[c, pl.ds(row0 + j * 128, 128), :])


_BN0 = 1000
_G0 = N // _BN0

def _dense0_body(x_ref, lin_ref, fw_ref, fb_ref, skip_ref, sfw_ref, sfb_ref,
                 cnt_ref, xl_ref, film_ref, base_ref):
    xb = x_ref[...]
    dims = (((1,), (1,)), ((), ()))
    sf = lax.dot_general(xb, sfw_ref[...], dims) + sfb_ref[...][None, :]
    beta_s = sf[:, :HID]
    gamma_s = sf[:, HID:]
    sk = lax.dot_general(xb, skip_ref[...], dims)
    base_ref[...] = jnp.maximum(gamma_s * sk + beta_s, 0.0)
    for r in range(R):
        xl_ref[r] = lax.dot_general(xb, lin_ref[r], dims)
        fl = (lax.dot_general(xb, fw_ref[r], dims)
              + fb_ref[pl.ds(r * 2 * HID, 2 * HID)][None, :])
        cnt_r = cnt_ref[0, r] + cnt_ref[0, R + r]
        inv = 1.0 / jnp.maximum(cnt_r, 1.0)
        film_ref[r] = fl * inv[:, None]


_dense0 = pl.pallas_call(
    _dense0_body,
    grid=(_G0,),
    in_specs=[
        pl.BlockSpec((_BN0, IN_C), lambda i: (i, 0)),
        pl.BlockSpec((R, HID, IN_C), lambda i: (0, 0, 0)),
        pl.BlockSpec((R, 2 * HID, IN_C), lambda i: (0, 0, 0)),
        pl.BlockSpec((R * 2 * HID,), lambda i: (0,)),
        pl.BlockSpec((HID, IN_C), lambda i: (0, 0)),
        pl.BlockSpec((2 * HID, IN_C), lambda i: (0, 0)),
        pl.BlockSpec((2 * HID,), lambda i: (0,)),
        pl.BlockSpec((1, 2 * R, _BN0), lambda i: (i, 0, 0)),
    ],
    out_specs=[
        pl.BlockSpec((R, _BN0, HID), lambda i: (0, i, 0)),
        pl.BlockSpec((R, _BN0, 2 * HID), lambda i: (0, i, 0)),
        pl.BlockSpec((_BN0, HID), lambda i: (i, 0)),
    ],
    out_shape=[
        jax.ShapeDtypeStruct((R, N, HID), _f32),
        jax.ShapeDtypeStruct((R, N, 2 * HID), _f32),
        jax.ShapeDtypeStruct((N, HID), _f32),
    ],
)


def _bn_body(acc_ref, base_ref, g_ref, b_ref, h_ref):
    h0 = base_ref[...] + acc_ref[0] + acc_ref[1]
    mu = jnp.mean(h0, axis=0)
    var = jnp.mean((h0 - mu[None, :]) ** 2, axis=0)
    scale = g_ref[...] * lax.rsqrt(var + 1e-5)
    h_ref[...] = (h0 - mu[None, :]) * scale[None, :] + b_ref[...][None, :]


_bn = pl.pallas_call(
    _bn_body,
    out_shape=jax.ShapeDtypeStruct((N, HID), _f32),
)


def _dense1_body(h_ref, lin_ref, fw_ref, fb_ref, skip_ref, sfw_ref, sfb_ref,
                 cnt_ref, t_ref, base_ref):
    hb = h_ref[...]
    dims = (((1,), (1,)), ((), ()))
    sf = lax.dot_general(hb, sfw_ref[...], dims) + sfb_ref[...][None, :]
    beta_s = sf[:, :OUT_C]
    gamma_s = sf[:, OUT_C:]
    sk = lax.dot_general(hb, skip_ref[...], dims)
    base = gamma_s * sk + beta_s
    for r in range(R):
        fl = (lax.dot_general(hb, fw_ref[r], dims)
              + fb_ref[pl.ds(r * 2 * OUT_C, 2 * OUT_C)][None, :])
        beta1 = fl[:, :OUT_C]
        gamma1 = fl[:, OUT_C:]
        cnt_r = cnt_ref[0, r] + cnt_ref[0, R + r]
        inv = 1.0 / jnp.maximum(cnt_r, 1.0)
        ind = jnp.minimum(cnt_r, 1.0)
        xl = lax.dot_general(hb, lin_ref[r], dims)
        t_ref[r] = jnp.concatenate([xl, gamma1 * inv[:, None]], axis=1)
        base = base + beta1 * ind[:, None]
    base_ref[...] = base


_dense1 = pl.pallas_call(
    _dense1_body,
    grid=(_G0,),
    in_specs=[
        pl.BlockSpec((_BN0, HID), lambda i: (i, 0)),
        pl.BlockSpec((R, OUT_C, HID), lambda i: (0, 0, 0)),
        pl.BlockSpec((R, 2 * OUT_C, HID), lambda i: (0, 0, 0)),
        pl.BlockSpec((R * 2 * OUT_C,), lambda i: (0,)),
        pl.BlockSpec((OUT_C, HID), lambda i: (0, 0)),
        pl.BlockSpec((2 * OUT_C, HID), lambda i: (0, 0)),
        pl.BlockSpec((2 * OUT_C,), lambda i: (0,)),
        pl.BlockSpec((1, 2 * R, _BN0), lambda i: (i, 0, 0)),
    ],
    out_specs=[
        pl.BlockSpec((R, _BN0, 2 * OUT_C), lambda i: (0, i, 0)),
        pl.BlockSpec((_BN0, OUT_C), lambda i: (i, 0)),
    ],
    out_shape=[
        jax.ShapeDtypeStruct((R, N, 2 * OUT_C), _f32),
        jax.ShapeDtypeStruct((N, OUT_C), _f32),
    ],
)


def _final_body(acc_ref, base_ref, out_ref):
    out_ref[...] = base_ref[...] + acc_ref[0] + acc_ref[1]


_final = pl.pallas_call(
    _final_body,
    out_shape=jax.ShapeDtypeStruct((N, OUT_C), _f32),
)


def kernel(x, edge_index, edge_type, c0_lin_w, c0_film_w, c0_film_b,
           c0_skip_w, c0_sfilm_w, c0_sfilm_b, bn_g, bn_b, c1_lin_w,
           c1_film_w, c1_film_b, c1_skip_w, c1_sfilm_w, c1_sfilm_b):
    src = edge_index[0]
    dst = edge_index[1]

    cntp = _sc_counts(dst, edge_type)
    cnt8 = (cntp[:, :RN].reshape(2 * R, _G0, _BN0)
            .transpose(1, 0, 2))

    xl0, film0p, base0 = _dense0(
        x, c0_lin_w, c0_film_w, c0_film_b.reshape(-1), c0_skip_w,
        c0_sfilm_w, c0_sfilm_b, cnt8)

    acc0p = _sc_edge0(xl0.reshape(RN, HID), film0p.reshape(RN, 2 * HID),
                      src, dst, edge_type)[:, :N, :]

    h = _bn(acc0p, base0, bn_g, bn_b)

    t1, base1 = _dense1(
        h, c1_lin_w, c1_film_w, c1_film_b.reshape(-1), c1_skip_w,
        c1_sfilm_w, c1_sfilm_b, cnt8)

    acc1p = _sc_edge1(t1.reshape(RN, 2 * OUT_C),
                      src, dst, edge_type)[:, :N, :OUT_C]

    return _final(acc1p, base1)


# speedup vs baseline: 16.9775x; 1.0407x over previous
import functools

import jax
import jax.numpy as jnp
from jax import lax
from jax.experimental import pallas as pl
from jax.experimental.pallas import tpu as pltpu
from jax.experimental.pallas import tpu_sc as plsc

N = 10000
E = 320000
IN_C = 128
HID = 128
OUT_C = 64
R = 4
RN = R * N

NC = 2
NS = 16
NW = NC * NS
EPW = E // NW
C = 80
SI = 2000
NSI = EPW // SI
NCI = SI // C

RNP = 40960
CNT_SLICE = RNP // NS

NROW = 10240
RPS = NROW // NS

_mesh = plsc.VectorSubcoreMesh(core_axis_name="c", subcore_axis_name="s")

_f32 = jnp.float32
_bf16 = jnp.bfloat16


def _zero_vec_loop(ref, rows, width):
    def body(i, _):
        for u in range(width // 16):
            ref[i, pl.ds(u * 16, 16)] = jnp.zeros((16,), _f32)
        return 0
    lax.fori_loop(0, rows, body, 0, unroll=False)


@functools.partial(
    pl.kernel,
    out_type=jax.ShapeDtypeStruct((NC, RNP), _f32),
    mesh=_mesh,
    scratch_types=[
        pltpu.VMEM_SHARED((RNP,), _f32),
        pltpu.VMEM((SI,), jnp.int32),
        pltpu.VMEM((SI,), jnp.int32),
        pltpu.VMEM((C,), jnp.int32),
        pltpu.VMEM((C,), _f32),
        pltpu.VMEM((CNT_SLICE,), _f32),
    ],
)
def _sc_counts(dst_hbm, typ_hbm, out_hbm, cnt_sh, dstv, typv, idxv, onesv, zb):
    c = lax.axis_index("c")
    s = lax.axis_index("s")
    wid = c * NS + s

    def zb_body(i, _):
        zb[pl.ds(i * 16, 16)] = jnp.zeros((16,), _f32)
        return 0
    lax.fori_loop(0, CNT_SLICE // 16, zb_body, 0, unroll=False)
    for j in range(C // 16):
        onesv[pl.ds(j * 16, 16)] = jnp.ones((16,), _f32)

    pltpu.sync_copy(zb, cnt_sh.at[pl.ds(s * CNT_SLICE, CNT_SLICE)])
    plsc.subcore_barrier()

    base = wid * EPW

    def superchunk(m, _):
        off = base + m * SI
        pltpu.sync_copy(dst_hbm.at[pl.ds(off, SI)], dstv)
        pltpu.sync_copy(typ_hbm.at[pl.ds(off, SI)], typv)

        def chunk(k, _):
            for j in range(C // 16):
                d16 = dstv[pl.ds(k * C + j * 16, 16)]
                r16 = typv[pl.ds(k * C + j * 16, 16)]
                idxv[pl.ds(j * 16, 16)] = r16 * N + d16
            pltpu.sync_copy(onesv, cnt_sh.at[idxv], add=True)
            return 0
        lax.fori_loop(0, NCI, chunk, 0, unroll=False)
        return 0
    lax.fori_loop(0, NSI, superchunk, 0, unroll=False)

    plsc.subcore_barrier()
    pltpu.sync_copy(cnt_sh.at[pl.ds(s * CNT_SLICE, CNT_SLICE)],
                    out_hbm.at[c, pl.ds(s * CNT_SLICE, CNT_SLICE)])


@functools.partial(
    pl.kernel,
    out_type=jax.ShapeDtypeStruct((NC, NROW, HID), _f32),
    mesh=_mesh,
    scratch_types=[
        pltpu.VMEM_SHARED((NROW, HID), _f32),
        pltpu.VMEM((SI,), jnp.int32),
        pltpu.VMEM((SI,), jnp.int32),
        pltpu.VMEM((SI,), jnp.int32),
        [pltpu.VMEM((C,), jnp.int32)] * 2,
        pltpu.VMEM((C,), jnp.int32),
        [pltpu.VMEM((C,), jnp.int32)] * 2,
        [pltpu.VMEM((C, HID), _f32)] * 2,
        pltpu.VMEM((C, 2 * HID), _f32),
        [pltpu.SemaphoreType.DMA] * 2,
        pltpu.SemaphoreType.DMA,
    ],
)
def _sc_edge0(xl_hbm, film_hbm, src_hbm, dst_hbm, typ_hbm, out_hbm,
              acc_sh, srcv, dstv, typv, isv, idv, sctv, xlb, fb,
              semx, semf):
    c = lax.axis_index("c")
    s = lax.axis_index("s")
    wid = c * NS + s
    row0 = s * RPS

    _zero_vec_loop(xlb[0], C, HID)
    for j in range(RPS // C):
        pltpu.sync_copy(xlb[0], acc_sh.at[pl.ds(row0 + j * C, C), :])
    plsc.subcore_barrier()

    base = wid * EPW

    def idx_for(k, p):
        for j in range(C // 16):
            s16 = srcv[pl.ds(k * C + j * 16, 16)]
            d16 = dstv[pl.ds(k * C + j * 16, 16)]
            r16 = typv[pl.ds(k * C + j * 16, 16)]
            isv[p][pl.ds(j * 16, 16)] = r16 * N + s16
            idv[pl.ds(j * 16, 16)] = r16 * N + d16
            sctv[p][pl.ds(j * 16, 16)] = d16

    def gathers(p):
        pltpu.async_copy(xl_hbm.at[isv[p]], xlb[p], semx[p])
        pltpu.async_copy(film_hbm.at[idv], fb, semf)

    def wait_gathers(p):
        pltpu.make_async_copy(film_hbm.at[idv], fb, semf).wait()
        pltpu.make_async_copy(xl_hbm.at[isv[p]], xlb[p], semx[p]).wait()

    def compute(p):
        def row(i, _):
            for u in range(HID // 16):
                x16 = xlb[p][i, pl.ds(u * 16, 16)]
                b16 = fb[i, pl.ds(u * 16, 16)]
                g16 = fb[i, pl.ds(HID + u * 16, 16)]
                xlb[p][i, pl.ds(u * 16, 16)] = jnp.maximum(
                    g16 * x16 + b16, 0.0)
            return 0
        lax.fori_loop(0, C, row, 0, unroll=False)

    def scatter(p):
        pltpu.sync_copy(xlb[p], acc_sh.at[sctv[p]], add=True)

    def half(k_cur, p, k_next):
        wait_gathers(p)
        compute(p)
        idx_for(k_next, 1 - p)
        gathers(1 - p)
        scatter(p)

    def superchunk(m, _):
        off = base + m * SI
        pltpu.sync_copy(src_hbm.at[pl.ds(off, SI)], srcv)
        pltpu.sync_copy(dst_hbm.at[pl.ds(off, SI)], dstv)
        pltpu.sync_copy(typ_hbm.at[pl.ds(off, SI)], typv)

        idx_for(0, 0)
        gathers(0)

        def pair(t, _):
            k0 = 2 * t
            half(k0, 0, k0 + 1)
            half(k0 + 1, 1, k0 + 2)
            return 0
        lax.fori_loop(0, NCI // 2, pair, 0, unroll=False)

        wait_gathers(0)
        compute(0)
        scatter(0)
        return 0
    lax.fori_loop(0, NSI, superchunk, 0, unroll=False)

    plsc.subcore_barrier()
    for j in range(RPS // 128):
        pltpu.sync_copy(acc_sh.at[pl.ds(row0 + j * 128, 128), :],
                        out_hbm.at[c, pl.ds(row0 + j * 128, 128), :])


@functools.partial(
    pl.kernel,
    out_type=jax.ShapeDtypeStruct((NC, NROW, 2 * OUT_C), _f32),
    mesh=_mesh,
    scratch_types=[
        pltpu.VMEM_SHARED((NROW, 2 * OUT_C), _f32),
        pltpu.VMEM((SI,), jnp.int32),
        pltpu.VMEM((SI,), jnp.int32),
        pltpu.VMEM((SI,), jnp.int32),
        pltpu.VMEM((C,), jnp.int32),
        pltpu.VMEM((C,), jnp.int32),
        pltpu.VMEM((C,), jnp.int32),
        pltpu.VMEM((C, 2 * OUT_C), _f32),
        pltpu.VMEM((C, 2 * OUT_C), _f32),
        pltpu.VMEM((C, 2 * OUT_C), _f32),
        pltpu.SemaphoreType.DMA,
        pltpu.SemaphoreType.DMA,
    ],
)
def _sc_edge1(t_hbm, src_hbm, dst_hbm, typ_hbm, out_hbm,
              acc_sh, srcv, dstv, typv, isv, idv, sctv, xlb, gb, mb,
              sem1, sem2):
    c = lax.axis_index("c")
    s = lax.axis_index("s")
    wid = c * NS + s
    row0 = s * RPS

    _zero_vec_loop(mb, C, 2 * OUT_C)
    for j in range(RPS // C):
        pltpu.sync_copy(mb, acc_sh.at[pl.ds(row0 + j * C, C), :])
    plsc.subcore_barrier()

    base = wid * EPW

    def superchunk(m, _):
        off = base + m * SI
        pltpu.sync_copy(src_hbm.at[pl.ds(off, SI)], srcv)
        pltpu.sync_copy(dst_hbm.at[pl.ds(off, SI)], dstv)
        pltpu.sync_copy(typ_hbm.at[pl.ds(off, SI)], typv)

        def chunk(k, _):
            for j in range(C // 16):
                s16 = srcv[pl.ds(k * C + j * 16, 16)]
                d16 = dstv[pl.ds(k * C + j * 16, 16)]
                r16 = typv[pl.ds(k * C + j * 16, 16)]
                isv[pl.ds(j * 16, 16)] = r16 * N + s16
                idv[pl.ds(j * 16, 16)] = r16 * N + d16
                sctv[pl.ds(j * 16, 16)] = d16
            cp1 = pltpu.async_copy(t_hbm.at[isv], xlb, sem1)
            cp2 = pltpu.async_copy(t_hbm.at[idv], gb, sem2)
            cp1.wait()
            cp2.wait()

            def row(i, _):
                for u in range(OUT_C // 16):
                    x16 = xlb[i, pl.ds(u * 16, 16)]
                    g16 = gb[i, pl.ds(OUT_C + u * 16, 16)]
                    mb[i, pl.ds(u * 16, 16)] = g16 * x16
                return 0
            lax.fori_loop(0, C, row, 0, unroll=False)

            pltpu.sync_copy(mb, acc_sh.at[sctv], add=True)
            return 0
        lax.fori_loop(0, NCI, chunk, 0, unroll=False)
        return 0
    lax.fori_loop(0, NSI, superchunk, 0, unroll=False)

    plsc.subcore_barrier()
    for j in range(RPS // 128):
        pltpu.sync_copy(acc_sh.at[pl.ds(row0 + j * 128, 128), :],
                        out_hbm.at[c, pl.ds(row0 + j * 128, 128), :])


_BN0 = 1000
_G0 = N // _BN0

def _dense0_body(x_ref, lin_ref, fw_ref, fb_ref, skip_ref, sfw_ref, sfb_ref,
                 cnt_ref, xl_ref, film_ref, base_ref):
    xb = x_ref[...]
    dims = (((1,), (1,)), ((), ()))
    sf = lax.dot_general(xb, sfw_ref[...], dims) + sfb_ref[...][None, :]
    beta_s = sf[:, :HID]
    gamma_s = sf[:, HID:]
    sk = lax.dot_general(xb, skip_ref[...], dims)
    base_ref[...] = jnp.maximum(gamma_s * sk + beta_s, 0.0)
    for r in range(R):
        xl_ref[r] = lax.dot_general(xb, lin_ref[r], dims)
        fl = (lax.dot_general(xb, fw_ref[r], dims)
              + fb_ref[pl.ds(r * 2 * HID, 2 * HID)][None, :])
        cnt_r = cnt_ref[0, r] + cnt_ref[0, R + r]
        inv = 1.0 / jnp.maximum(cnt_r, 1.0)
        film_ref[r] = fl * inv[:, None]


_dense0 = pl.pallas_call(
    _dense0_body,
    grid=(_G0,),
    in_specs=[
        pl.BlockSpec((_BN0, IN_C), lambda i: (i, 0)),
        pl.BlockSpec((R, HID, IN_C), lambda i: (0, 0, 0)),
        pl.BlockSpec((R, 2 * HID, IN_C), lambda i: (0, 0, 0)),
        pl.BlockSpec((R * 2 * HID,), lambda i: (0,)),
        pl.BlockSpec((HID, IN_C), lambda i: (0, 0)),
        pl.BlockSpec((2 * HID, IN_C), lambda i: (0, 0)),
        pl.BlockSpec((2 * HID,), lambda i: (0,)),
        pl.BlockSpec((1, 2 * R, _BN0), lambda i: (i, 0, 0)),
    ],
    out_specs=[
        pl.BlockSpec((R, _BN0, HID), lambda i: (0, i, 0)),
        pl.BlockSpec((R, _BN0, 2 * HID), lambda i: (0, i, 0)),
        pl.BlockSpec((_BN0, HID), lambda i: (i, 0)),
    ],
    out_shape=[
        jax.ShapeDtypeStruct((R, N, HID), _f32),
        jax.ShapeDtypeStruct((R, N, 2 * HID), _f32),
        jax.ShapeDtypeStruct((N, HID), _f32),
    ],
)


def _bn_body(acc_ref, base_ref, g_ref, b_ref, h_ref):
    h0 = base_ref[...] + acc_ref[0] + acc_ref[1]
    mu = jnp.mean(h0, axis=0)
    var = jnp.mean((h0 - mu[None, :]) ** 2, axis=0)
    scale = g_ref[...] * lax.rsqrt(var + 1e-5)
    h_ref[...] = (h0 - mu[None, :]) * scale[None, :] + b_ref[...][None, :]


_bn = pl.pallas_call(
    _bn_body,
    out_shape=jax.ShapeDtypeStruct((N, HID), _f32),
)


def _dense1_body(h_ref, lin_ref, fw_ref, fb_ref, skip_ref, sfw_ref, sfb_ref,
                 cnt_ref, t_ref, base_ref):
    hb = h_ref[...]
    dims = (((1,), (1,)), ((), ()))
    sf = lax.dot_general(hb, sfw_ref[...], dims) + sfb_ref[...][None, :]
    beta_s = sf[:, :OUT_C]
    gamma_s = sf[:, OUT_C:]
    sk = lax.dot_general(hb, skip_ref[...], dims)
    base = gamma_s * sk + beta_s
    for r in range(R):
        fl = (lax.dot_general(hb, fw_ref[r], dims)
              + fb_ref[pl.ds(r * 2 * OUT_C, 2 * OUT_C)][None, :])
        beta1 = fl[:, :OUT_C]
        gamma1 = fl[:, OUT_C:]
        cnt_r = cnt_ref[0, r] + cnt_ref[0, R + r]
        inv = 1.0 / jnp.maximum(cnt_r, 1.0)
        ind = jnp.minimum(cnt_r, 1.0)
        xl = lax.dot_general(hb, lin_ref[r], dims)
        t_ref[r] = jnp.concatenate([xl, gamma1 * inv[:, None]], axis=1)
        base = base + beta1 * ind[:, None]
    base_ref[...] = base


_dense1 = pl.pallas_call(
    _dense1_body,
    grid=(_G0,),
    in_specs=[
        pl.BlockSpec((_BN0, HID), lambda i: (i, 0)),
        pl.BlockSpec((R, OUT_C, HID), lambda i: (0, 0, 0)),
        pl.BlockSpec((R, 2 * OUT_C, HID), lambda i: (0, 0, 0)),
        pl.BlockSpec((R * 2 * OUT_C,), lambda i: (0,)),
        pl.BlockSpec((OUT_C, HID), lambda i: (0, 0)),
        pl.BlockSpec((2 * OUT_C, HID), lambda i: (0, 0)),
        pl.BlockSpec((2 * OUT_C,), lambda i: (0,)),
        pl.BlockSpec((1, 2 * R, _BN0), lambda i: (i, 0, 0)),
    ],
    out_specs=[
        pl.BlockSpec((R, _BN0, 2 * OUT_C), lambda i: (0, i, 0)),
        pl.BlockSpec((_BN0, OUT_C), lambda i: (i, 0)),
    ],
    out_shape=[
        jax.ShapeDtypeStruct((R, N, 2 * OUT_C), _f32),
        jax.ShapeDtypeStruct((N, OUT_C), _f32),
    ],
)


def _final_body(acc_ref, base_ref, out_ref):
    out_ref[...] = base_ref[...] + acc_ref[0] + acc_ref[1]


_final = pl.pallas_call(
    _final_body,
    out_shape=jax.ShapeDtypeStruct((N, OUT_C), _f32),
)


def kernel(x, edge_index, edge_type, c0_lin_w, c0_film_w, c0_film_b,
           c0_skip_w, c0_sfilm_w, c0_sfilm_b, bn_g, bn_b, c1_lin_w,
           c1_film_w, c1_film_b, c1_skip_w, c1_sfilm_w, c1_sfilm_b):
    src = edge_index[0]
    dst = edge_index[1]

    cntp = _sc_counts(dst, edge_type)
    cnt8 = (cntp[:, :RN].reshape(2 * R, _G0, _BN0)
            .transpose(1, 0, 2))

    xl0, film0p, base0 = _dense0(
        x, c0_lin_w, c0_film_w, c0_film_b.reshape(-1), c0_skip_w,
        c0_sfilm_w, c0_sfilm_b, cnt8)

    acc0p = _sc_edge0(xl0.reshape(RN, HID), film0p.reshape(RN, 2 * HID),
                      src, dst, edge_type)[:, :N, :]

    h = _bn(acc0p, base0, bn_g, bn_b)

    t1, base1 = _dense1(
        h, c1_lin_w, c1_film_w, c1_film_b.reshape(-1), c1_skip_w,
        c1_sfilm_w, c1_sfilm_b, cnt8)

    acc1p = _sc_edge1(t1.reshape(RN, 2 * OUT_C),
                      src, dst, edge_type)[:, :N, :OUT_C]

    return _final(acc1p, base1)


# speedup vs baseline: 17.0353x; 1.0034x over previous
import functools

import jax
import jax.numpy as jnp
from jax import lax
from jax.experimental import pallas as pl
from jax.experimental.pallas import tpu as pltpu
from jax.experimental.pallas import tpu_sc as plsc

N = 10000
E = 320000
IN_C = 128
HID = 128
OUT_C = 64
R = 4
RN = R * N

NC = 2
NS = 16
NW = NC * NS
EPW = E // NW
C = 80
SI = 2000
NSI = EPW // SI
NCI = SI // C

RNP = 40960
CNT_SLICE = RNP // NS

NROW = 10240
RPS = NROW // NS

_mesh = plsc.VectorSubcoreMesh(core_axis_name="c", subcore_axis_name="s")

_f32 = jnp.float32
_bf16 = jnp.bfloat16


def _zero_vec_loop(ref, rows, width):
    def body(i, _):
        for u in range(width // 16):
            ref[i, pl.ds(u * 16, 16)] = jnp.zeros((16,), _f32)
        return 0
    lax.fori_loop(0, rows, body, 0, unroll=False)


@functools.partial(
    pl.kernel,
    out_type=jax.ShapeDtypeStruct((NC, RNP), _f32),
    mesh=_mesh,
    scratch_types=[
        pltpu.VMEM_SHARED((RNP,), _f32),
        pltpu.VMEM((SI,), jnp.int32),
        pltpu.VMEM((SI,), jnp.int32),
        pltpu.VMEM((C,), jnp.int32),
        pltpu.VMEM((C,), _f32),
        pltpu.VMEM((CNT_SLICE,), _f32),
    ],
)
def _sc_counts(dst_hbm, typ_hbm, out_hbm, cnt_sh, dstv, typv, idxv, onesv, zb):
    c = lax.axis_index("c")
    s = lax.axis_index("s")
    wid = c * NS + s

    def zb_body(i, _):
        zb[pl.ds(i * 16, 16)] = jnp.zeros((16,), _f32)
        return 0
    lax.fori_loop(0, CNT_SLICE // 16, zb_body, 0, unroll=False)
    for j in range(C // 16):
        onesv[pl.ds(j * 16, 16)] = jnp.ones((16,), _f32)

    pltpu.sync_copy(zb, cnt_sh.at[pl.ds(s * CNT_SLICE, CNT_SLICE)])
    plsc.subcore_barrier()

    base = wid * EPW

    def superchunk(m, _):
        off = base + m * SI
        pltpu.sync_copy(dst_hbm.at[pl.ds(off, SI)], dstv)
        pltpu.sync_copy(typ_hbm.at[pl.ds(off, SI)], typv)

        def chunk(k, _):
            for j in range(C // 16):
                d16 = dstv[pl.ds(k * C + j * 16, 16)]
                r16 = typv[pl.ds(k * C + j * 16, 16)]
                idxv[pl.ds(j * 16, 16)] = r16 * N + d16
            pltpu.sync_copy(onesv, cnt_sh.at[idxv], add=True)
            return 0
        lax.fori_loop(0, NCI, chunk, 0, unroll=False)
        return 0
    lax.fori_loop(0, NSI, superchunk, 0, unroll=False)

    plsc.subcore_barrier()
    pltpu.sync_copy(cnt_sh.at[pl.ds(s * CNT_SLICE, CNT_SLICE)],
                    out_hbm.at[c, pl.ds(s * CNT_SLICE, CNT_SLICE)])


@functools.partial(
    pl.kernel,
    out_type=jax.ShapeDtypeStruct((NC, NROW, HID), _f32),
    mesh=_mesh,
    scratch_types=[
        pltpu.VMEM_SHARED((NROW, HID), _f32),
        pltpu.VMEM((SI,), jnp.int32),
        pltpu.VMEM((SI,), jnp.int32),
        pltpu.VMEM((SI,), jnp.int32),
        [pltpu.VMEM((C,), jnp.int32)] * 2,
        pltpu.VMEM((C,), jnp.int32),
        [pltpu.VMEM((C,), jnp.int32)] * 2,
        [pltpu.VMEM((C, HID), _f32)] * 2,
        pltpu.VMEM((C, 2 * HID), _f32),
        [pltpu.SemaphoreType.DMA] * 2,
        pltpu.SemaphoreType.DMA,
        [pltpu.SemaphoreType.DMA] * 2,
    ],
)
def _sc_edge0(xl_hbm, film_hbm, src_hbm, dst_hbm, typ_hbm, out_hbm,
              acc_sh, srcv, dstv, typv, isv, idv, sctv, xlb, fb,
              semx, semf, sems):
    c = lax.axis_index("c")
    s = lax.axis_index("s")
    wid = c * NS + s
    row0 = s * RPS

    _zero_vec_loop(xlb[0], C, HID)
    _zero_vec_loop(xlb[1], C, HID)
    for j in range(C // 16):
        sctv[0][pl.ds(j * 16, 16)] = jnp.zeros((16,), jnp.int32)
        sctv[1][pl.ds(j * 16, 16)] = jnp.zeros((16,), jnp.int32)
    for j in range(RPS // C):
        pltpu.sync_copy(xlb[0], acc_sh.at[pl.ds(row0 + j * C, C), :])
    plsc.subcore_barrier()

    base = wid * EPW

    def idx_for(k, p):
        for j in range(C // 16):
            s16 = srcv[pl.ds(k * C + j * 16, 16)]
            d16 = dstv[pl.ds(k * C + j * 16, 16)]
            r16 = typv[pl.ds(k * C + j * 16, 16)]
            isv[p][pl.ds(j * 16, 16)] = r16 * N + s16
            idv[pl.ds(j * 16, 16)] = r16 * N + d16
            sctv[p][pl.ds(j * 16, 16)] = d16

    def gathers(p):
        pltpu.async_copy(xl_hbm.at[isv[p]], xlb[p], semx[p])
        pltpu.async_copy(film_hbm.at[idv], fb, semf)

    def wait_gathers(p):
        pltpu.make_async_copy(film_hbm.at[idv], fb, semf).wait()
        pltpu.make_async_copy(xl_hbm.at[isv[p]], xlb[p], semx[p]).wait()

    def compute(p):
        def row(i, _):
            for u in range(HID // 16):
                x16 = xlb[p][i, pl.ds(u * 16, 16)]
                b16 = fb[i, pl.ds(u * 16, 16)]
                g16 = fb[i, pl.ds(HID + u * 16, 16)]
                xlb[p][i, pl.ds(u * 16, 16)] = jnp.maximum(
                    g16 * x16 + b16, 0.0)
            return 0
        lax.fori_loop(0, C, row, 0, unroll=2)

    def scatter(p):
        pltpu.async_copy(xlb[p], acc_sh.at[sctv[p]], sems[p], add=True)

    def wait_scatter(p):
        pltpu.make_async_copy(xlb[p], acc_sh.at[sctv[p]], sems[p]).wait()

    def half(k_cur, p, k_next):
        wait_gathers(p)
        compute(p)
        wait_scatter(1 - p)
        idx_for(k_next, 1 - p)
        gathers(1 - p)
        scatter(p)

    def superchunk(m, _):
        off = base + m * SI
        pltpu.sync_copy(src_hbm.at[pl.ds(off, SI)], srcv)
        pltpu.sync_copy(dst_hbm.at[pl.ds(off, SI)], dstv)
        pltpu.sync_copy(typ_hbm.at[pl.ds(off, SI)], typv)

        idx_for(0, 0)
        gathers(0)
        _zero_vec_loop(xlb[1], C, HID)
        for j in range(C // 16):
            sctv[1][pl.ds(j * 16, 16)] = jnp.zeros((16,), jnp.int32)
        scatter(1)

        def pair(t, _):
            k0 = 2 * t
            half(k0, 0, k0 + 1)
            half(k0 + 1, 1, k0 + 2)
            return 0
        lax.fori_loop(0, NCI // 2, pair, 0, unroll=False)

        wait_gathers(0)
        compute(0)
        wait_scatter(1)
        scatter(0)
        wait_scatter(0)
        return 0
    lax.fori_loop(0, NSI, superchunk, 0, unroll=False)

    plsc.subcore_barrier()
    for j in range(RPS // 128):
        pltpu.sync_copy(acc_sh.at[pl.ds(row0 + j * 128, 128), :],
                        out_hbm.at[c, pl.ds(row0 + j * 128, 128), :])


@functools.partial(
    pl.kernel,
    out_type=jax.ShapeDtypeStruct((NC, NROW, 2 * OUT_C), _f32),
    mesh=_mesh,
    scratch_types=[
        pltpu.VMEM_SHARED((NROW, 2 * OUT_C), _f32),
        pltpu.VMEM((SI,), jnp.int32),
        pltpu.VMEM((SI,), jnp.int32),
        pltpu.VMEM((SI,), jnp.int32),
        pltpu.VMEM((C,), jnp.int32),
        pltpu.VMEM((C,), jnp.int32),
        pltpu.VMEM((C,), jnp.int32),
        pltpu.VMEM((C, 2 * OUT_C), _f32),
        pltpu.VMEM((C, 2 * OUT_C), _f32),
        pltpu.VMEM((C, 2 * OUT_C), _f32),
        pltpu.SemaphoreType.DMA,
        pltpu.SemaphoreType.DMA,
    ],
)
def _sc_edge1(t_hbm, src_hbm, dst_hbm, typ_hbm, out_hbm,
              acc_sh, srcv, dstv, typv, isv, idv, sctv, xlb, gb, mb,
              sem1, sem2):
    c = lax.axis_index("c")
    s = lax.axis_index("s")
    wid = c * NS + s
    row0 = s * RPS

    _zero_vec_loop(mb, C, 2 * OUT_C)
    for j in range(RPS // C):
        pltpu.sync_copy(mb, acc_sh.at[pl.ds(row0 + j * C, C), :])
    plsc.subcore_barrier()

    base = wid * EPW

    def superchunk(m, _):
        off = base + m * SI
        pltpu.sync_copy(src_hbm.at[pl.ds(off, SI)], srcv)
        pltpu.sync_copy(dst_hbm.at[pl.ds(off, SI)], dstv)
        pltpu.sync_copy(typ_hbm.at[pl.ds(off, SI)], typv)

        def chunk(k, _):
            for j in range(C // 16):
                s16 = srcv[pl.ds(k * C + j * 16, 16)]
                d16 = dstv[pl.ds(k * C + j * 16, 16)]
                r16 = typv[pl.ds(k * C + j * 16, 16)]
                isv[pl.ds(j * 16, 16)] = r16 * N + s16
                idv[pl.ds(j * 16, 16)] = r16 * N + d16
                sctv[pl.ds(j * 16, 16)] = d16
            cp1 = pltpu.async_copy(t_hbm.at[isv], xlb, sem1)
            cp2 = pltpu.async_copy(t_hbm.at[idv], gb, sem2)
            cp1.wait()
            cp2.wait()

            def row(i, _):
                for u in range(OUT_C // 16):
                    x16 = xlb[i, pl.ds(u * 16, 16)]
                    g16 = gb[i, pl.ds(OUT_C + u * 16, 16)]
                    mb[i, pl.ds(u * 16, 16)] = g16 * x16
                return 0
            lax.fori_loop(0, C, row, 0, unroll=False)

            pltpu.sync_copy(mb, acc_sh.at[sctv], add=True)
            return 0
        lax.fori_loop(0, NCI, chunk, 0, unroll=False)
        return 0
    lax.fori_loop(0, NSI, superchunk, 0, unroll=False)

    plsc.subcore_barrier()
    for j in range(RPS // 128):
        pltpu.sync_copy(acc_sh.at[pl.ds(row0 + j * 128, 128), :],
                        out_hbm.at[c, pl.ds(row0 + j * 128, 128), :])


_BN0 = 1000
_G0 = N // _BN0

def _dense0_body(x_ref, lin_ref, fw_ref, fb_ref, skip_ref, sfw_ref, sfb_ref,
                 cnt_ref, xl_ref, film_ref, base_ref):
    xb = x_ref[...]
    dims = (((1,), (1,)), ((), ()))
    sf = lax.dot_general(xb, sfw_ref[...], dims) + sfb_ref[...][None, :]
    beta_s = sf[:, :HID]
    gamma_s = sf[:, HID:]
    sk = lax.dot_general(xb, skip_ref[...], dims)
    base_ref[...] = jnp.maximum(gamma_s * sk + beta_s, 0.0)
    for r in range(R):
        xl_ref[r] = lax.dot_general(xb, lin_ref[r], dims)
        fl = (lax.dot_general(xb, fw_ref[r], dims)
              + fb_ref[pl.ds(r * 2 * HID, 2 * HID)][None, :])
        cnt_r = cnt_ref[0, r] + cnt_ref[0, R + r]
        inv = 1.0 / jnp.maximum(cnt_r, 1.0)
        film_ref[r] = fl * inv[:, None]


_dense0 = pl.pallas_call(
    _dense0_body,
    grid=(_G0,),
    in_specs=[
        pl.BlockSpec((_BN0, IN_C), lambda i: (i, 0)),
        pl.BlockSpec((R, HID, IN_C), lambda i: (0, 0, 0)),
        pl.BlockSpec((R, 2 * HID, IN_C), lambda i: (0, 0, 0)),
        pl.BlockSpec((R * 2 * HID,), lambda i: (0,)),
        pl.BlockSpec((HID, IN_C), lambda i: (0, 0)),
        pl.BlockSpec((2 * HID, IN_C), lambda i: (0, 0)),
        pl.BlockSpec((2 * HID,), lambda i: (0,)),
        pl.BlockSpec((1, 2 * R, _BN0), lambda i: (i, 0, 0)),
    ],
    out_specs=[
        pl.BlockSpec((R, _BN0, HID), lambda i: (0, i, 0)),
        pl.BlockSpec((R, _BN0, 2 * HID), lambda i: (0, i, 0)),
        pl.BlockSpec((_BN0, HID), lambda i: (i, 0)),
    ],
    out_shape=[
        jax.ShapeDtypeStruct((R, N, HID), _f32),
        jax.ShapeDtypeStruct((R, N, 2 * HID), _f32),
        jax.ShapeDtypeStruct((N, HID), _f32),
    ],
)


def _bn_body(acc_ref, base_ref, g_ref, b_ref, h_ref):
    h0 = base_ref[...] + acc_ref[0] + acc_ref[1]
    mu = jnp.mean(h0, axis=0)
    var = jnp.mean((h0 - mu[None, :]) ** 2, axis=0)
    scale = g_ref[...] * lax.rsqrt(var + 1e-5)
    h_ref[...] = (h0 - mu[None, :]) * scale[None, :] + b_ref[...][None, :]


_bn = pl.pallas_call(
    _bn_body,
    out_shape=jax.ShapeDtypeStruct((N, HID), _f32),
)


def _dense1_body(h_ref, lin_ref, fw_ref, fb_ref, skip_ref, sfw_ref, sfb_ref,
                 cnt_ref, t_ref, base_ref):
    hb = h_ref[...]
    dims = (((1,), (1,)), ((), ()))
    sf = lax.dot_general(hb, sfw_ref[...], dims) + sfb_ref[...][None, :]
    beta_s = sf[:, :OUT_C]
    gamma_s = sf[:, OUT_C:]
    sk = lax.dot_general(hb, skip_ref[...], dims)
    base = gamma_s * sk + beta_s
    for r in range(R):
        fl = (lax.dot_general(hb, fw_ref[r], dims)
              + fb_ref[pl.ds(r * 2 * OUT_C, 2 * OUT_C)][None, :])
        beta1 = fl[:, :OUT_C]
        gamma1 = fl[:, OUT_C:]
        cnt_r = cnt_ref[0, r] + cnt_ref[0, R + r]
        inv = 1.0 / jnp.maximum(cnt_r, 1.0)
        ind = jnp.minimum(cnt_r, 1.0)
        xl = lax.dot_general(hb, lin_ref[r], dims)
        t_ref[r] = jnp.concatenate([xl, gamma1 * inv[:, None]], axis=1)
        base = base + beta1 * ind[:, None]
    base_ref[...] = base


_dense1 = pl.pallas_call(
    _dense1_body,
    grid=(_G0,),
    in_specs=[
        pl.BlockSpec((_BN0, HID), lambda i: (i, 0)),
        pl.BlockSpec((R, OUT_C, HID), lambda i: (0, 0, 0)),
        pl.BlockSpec((R, 2 * OUT_C, HID), lambda i: (0, 0, 0)),
        pl.BlockSpec((R * 2 * OUT_C,), lambda i: (0,)),
        pl.BlockSpec((OUT_C, HID), lambda i: (0, 0)),
        pl.BlockSpec((2 * OUT_C, HID), lambda i: (0, 0)),
        pl.BlockSpec((2 * OUT_C,), lambda i: (0,)),
        pl.BlockSpec((1, 2 * R, _BN0), lambda i: (i, 0, 0)),
    ],
    out_specs=[
        pl.BlockSpec((R, _BN0, 2 * OUT_C), lambda i: (0, i, 0)),
        pl.BlockSpec((_BN0, OUT_C), lambda i: (i, 0)),
    ],
    out_shape=[
        jax.ShapeDtypeStruct((R, N, 2 * OUT_C), _f32),
        jax.ShapeDtypeStruct((N, OUT_C), _f32),
    ],
)


def _final_body(acc_ref, base_ref, out_ref):
    out_ref[...] = base_ref[...] + acc_ref[0] + acc_ref[1]


_final = pl.pallas_call(
    _final_body,
    out_shape=jax.ShapeDtypeStruct((N, OUT_C), _f32),
)


def kernel(x, edge_index, edge_type, c0_lin_w, c0_film_w, c0_film_b,
           c0_skip_w, c0_sfilm_w, c0_sfilm_b, bn_g, bn_b, c1_lin_w,
           c1_film_w, c1_film_b, c1_skip_w, c1_sfilm_w, c1_sfilm_b):
    src = edge_index[0]
    dst = edge_index[1]

    cntp = _sc_counts(dst, edge_type)
    cnt8 = (cntp[:, :RN].reshape(2 * R, _G0, _BN0)
            .transpose(1, 0, 2))

    xl0, film0p, base0 = _dense0(
        x, c0_lin_w, c0_film_w, c0_film_b.reshape(-1), c0_skip_w,
        c0_sfilm_w, c0_sfilm_b, cnt8)

    acc0p = _sc_edge0(xl0.reshape(RN, HID), film0p.reshape(RN, 2 * HID),
                      src, dst, edge_type)[:, :N, :]

    h = _bn(acc0p, base0, bn_g, bn_b)

    t1, base1 = _dense1(
        h, c1_lin_w, c1_film_w, c1_film_b.reshape(-1), c1_skip_w,
        c1_sfilm_w, c1_sfilm_b, cnt8)

    acc1p = _sc_edge1(t1.reshape(RN, 2 * OUT_C),
                      src, dst, edge_type)[:, :N, :OUT_C]

    return _final(acc1p, base1)
